# baseline (speedup 1.0000x reference)
import functools
import math

import jax
import jax.numpy as jnp
from jax import lax
from jax.experimental import pallas as pl
from jax.experimental.pallas import tpu as pltpu
from jax.experimental.pallas import tpu_sc as plsc

N = 3200000
S = 100000
EPS = 1e-05
PI = math.pi
TWO_PI = 2.0 * math.pi
INV_TWO_PI = 1.0 / TWO_PI

NW = 32
SUB = 128
NSUB = N // SUB
K = 20
CH = K * SUB
NOUT = NSUB // K
PAIRS = NOUT // 2
_P_BASE = PAIRS // NW
_P_REM = PAIRS - _P_BASE * NW
SP = 100096
STRIPE = SP // 16
MSTR = SP // 32

_mesh = plsc.VectorSubcoreMesh(core_axis_name="c", subcore_axis_name="s")
_sc_params = pltpu.CompilerParams(needs_layout_passes=False,
                                  use_tc_tiling_on_sc=False)


def _worker_span(w):
    base = w * _P_BASE + jnp.minimum(w, _P_REM)
    cnt = jnp.where(w < _P_REM, _P_BASE + 1, _P_BASE)
    return base, cnt


def _col(v):
    return jnp.full((16,), v, jnp.int32)


@functools.partial(
    pl.kernel,
    out_type=jax.ShapeDtypeStruct((2, SP, 8), jnp.float32),
    mesh=_mesh,
    scratch_types=[
        pltpu.VMEM((CH,), jnp.int32)] * 2 + [pltpu.VMEM((CH,), jnp.float32)] * 12 + [
        pltpu.VMEM((CH, 8), jnp.float32)] * 2 + [
        pltpu.VMEM_SHARED((SP, 8), jnp.float32),
        pltpu.SemaphoreType.DMA,
        pltpu.SemaphoreType.DMA,
        pltpu.SemaphoreType.DMA,
    ],
    compiler_params=_sc_params,
)
def _accum(ev0, ev1, ev2, er0, er1, er2, idx_hbm, zeros_hbm, pout_hbm,
           idx_a, idx_b, a0, a1, a2, a3, a4, a5, c0, c1, c2, c3, c4, c5,
           aug_a, aug_b, sh, sem_a, sem_b, sem_sc):
    c = lax.axis_index("c")
    s = lax.axis_index("s")
    w = c * 16 + s
    lanes = lax.iota(jnp.int32, 16)
    pltpu.sync_copy(zeros_hbm.at[pl.ds(s * STRIPE, STRIPE)],
                    sh.at[pl.ds(s * STRIPE, STRIPE)])
    for aug in (aug_a, aug_b):
        @pl.loop(0, CH // 16)
        def _(g):
            r = g * 16 + lanes
            plsc.store_scatter(aug, [r, _col(6)], jnp.full((16,), 1.0, jnp.float32))
            plsc.store_scatter(aug, [r, _col(7)], jnp.full((16,), 0.0, jnp.float32))

    plsc.subcore_barrier()
    base, cnt = _worker_span(w)
    bufs_a = (a0, a1, a2, a3, a4, a5)
    bufs_b = (c0, c1, c2, c3, c4, c5)
    srcs = (ev0, ev1, ev2, er0, er1, er2)

    def _assemble(bufs, aug):
        @pl.loop(0, CH // 16)
        def _(g):
            r = g * 16 + lanes
            sl = pl.ds(g * 16, 16)
            for col, b in enumerate(bufs):
                plsc.store_scatter(aug, [r, _col(col)], b[sl])

    @pl.loop(base, base + cnt)
    def _(u):
        row_a = (2 * u) * CH
        row_b = row_a + CH
        sl_a = pl.ds(row_a, CH)
        sl_b = pl.ds(row_b, CH)
        da = [pltpu.async_copy(idx_hbm.at[sl_a], idx_a, sem_a)]
        da += [pltpu.async_copy(src.at[sl_a], b, sem_a)
               for b, src in zip(bufs_a, srcs)]
        db = [pltpu.async_copy(idx_hbm.at[sl_b], idx_b, sem_b)]
        db += [pltpu.async_copy(src.at[sl_b], b, sem_b)
               for b, src in zip(bufs_b, srcs)]
        for d in da:
            d.wait()
        _assemble(bufs_a, aug_a)
        scs = []
        for k in range(K):
            sl_k = pl.ds(k * SUB, SUB)
            scs.append(pltpu.async_copy(aug_a.at[sl_k],
                                        sh.at[idx_a.at[sl_k]],
                                        sem_sc, add=True))
        for d in db:
            d.wait()
        _assemble(bufs_b, aug_b)
        for k in range(K):
            sl_k = pl.ds(k * SUB, SUB)
            scs.append(pltpu.async_copy(aug_b.at[sl_k],
                                        sh.at[idx_b.at[sl_k]],
                                        sem_sc, add=True))
        for d in scs:
            d.wait()

    plsc.subcore_barrier()
    pltpu.sync_copy(sh.at[pl.ds(s * STRIPE, STRIPE)],
                    pout_hbm.at[c].at[pl.ds(s * STRIPE, STRIPE)])


def _merge_body(p_ref, o_ref):
    p = p_ref[...]
    sm = p[0] + p[1]
    cnt = jnp.clip(sm[:, 6:7], 1.0, None)
    o_ref[...] = sm[:, :6] / cnt


def _merge(partials):
    bs = 3128
    return pl.pallas_call(
        _merge_body,
        grid=(SP // bs,),
        in_specs=[pl.BlockSpec((2, bs, 8), lambda i: (0, i, 0))],
        out_specs=pl.BlockSpec((bs, 6), lambda i: (i, 0)),
        out_shape=jax.ShapeDtypeStruct((SP, 6), jnp.float32),
    )(partials)


@functools.partial(
    pl.kernel,
    out_type=[jax.ShapeDtypeStruct((N,), jnp.float32)] * 6,
    mesh=_mesh,
    scratch_types=[
        pltpu.VMEM((CH,), jnp.int32)] * 2 + [pltpu.VMEM((CH,), jnp.float32)] * 12 + [
        pltpu.VMEM((CH, 6), jnp.float32)] * 2 + [
        pltpu.SemaphoreType.DMA,
        pltpu.SemaphoreType.DMA,
        pltpu.SemaphoreType.DMA,
        pltpu.SemaphoreType.DMA,
        pltpu.SemaphoreType.DMA,
    ],
    compiler_params=_sc_params,
)
def _center(table, idx_hbm, ev0, ev1, ev2, er0, er1, er2,
            oev0, oev1, oev2, oer0, oer1, oer2,
            idx_a, idx_b, a0, a1, a2, a3, a4, a5, c0, c1, c2, c3, c4, c5,
            rows_a, rows_b, sem_a, sem_b, sem_ga, sem_gb, sem_out):
    c = lax.axis_index("c")
    s = lax.axis_index("s")
    w = c * 16 + s
    lanes = lax.iota(jnp.int32, 16)
    base, cnt = _worker_span(w)
    bufs_a = (a0, a1, a2, a3, a4, a5)
    bufs_b = (c0, c1, c2, c3, c4, c5)
    srcs = (ev0, ev1, ev2, er0, er1, er2)
    outs = (oev0, oev1, oev2, oer0, oer1, oer2)

    def _subtract(bufs, rows):
        @pl.loop(0, CH // 16)
        def _(g):
            r = g * 16 + lanes
            sl = pl.ds(g * 16, 16)
            for col, b in enumerate(bufs):
                b[sl] = b[sl] - plsc.load_gather(rows, [r, _col(col)])

    @pl.loop(base, base + cnt)
    def _(u):
        row_a = (2 * u) * CH
        row_b = row_a + CH
        sl_a = pl.ds(row_a, CH)
        sl_b = pl.ds(row_b, CH)
        dia = pltpu.async_copy(idx_hbm.at[sl_a], idx_a, sem_a)
        dib = pltpu.async_copy(idx_hbm.at[sl_b], idx_b, sem_b)
        da = [pltpu.async_copy(src.at[sl_a], b, sem_a)
              for b, src in zip(bufs_a, srcs)]
        db = [pltpu.async_copy(src.at[sl_b], b, sem_b)
              for b, src in zip(bufs_b, srcs)]
        dia.wait()
        ga = [pltpu.async_copy(table.at[idx_a.at[pl.ds(k * SUB, SUB)]],
                               rows_a.at[pl.ds(k * SUB, SUB)], sem_ga)
              for k in range(K)]
        dib.wait()
        gb = [pltpu.async_copy(table.at[idx_b.at[pl.ds(k * SUB, SUB)]],
                               rows_b.at[pl.ds(k * SUB, SUB)], sem_gb)
              for k in range(K)]
        for d in da:
            d.wait()
        for d in ga:
            d.wait()
        _subtract(bufs_a, rows_a)
        oda = [pltpu.async_copy(b, dst.at[sl_a], sem_out)
               for b, dst in zip(bufs_a, outs)]
        for d in db:
            d.wait()
        for d in gb:
            d.wait()
        _subtract(bufs_b, rows_b)
        odb = [pltpu.async_copy(b, dst.at[sl_b], sem_out)
               for b, dst in zip(bufs_b, outs)]
        for d in oda:
            d.wait()
        for d in odb:
            d.wait()


_GRID = 25
_BLK = N // _GRID


def _wrap_pi(x):
    return jnp.remainder(x + PI, TWO_PI) - PI


def _dense_body(t_ref, f00, f01, f02, v00, v01, v02, e0, e1, e2, r0, r1, r2,
                ft0, ft1, ft2, vt0, vt1, vt2, rt0, rt1, rt2):
    tt = 2.0 * t_ref[...]
    e = jnp.exp(-tt)
    sigma_v = jnp.sqrt(jnp.clip(1.0 - e * e, EPS, None))
    coeff = (1.0 - e) / (1.0 + e)
    sig_r = jnp.sqrt(jnp.clip(2.0 * tt + 8.0 / (1.0 + jnp.exp(tt)) - 4.0, EPS, None))
    for f0c, v0c, evc, erc, ftc, vtc, rtc in (
            (f00, v00, e0, r0, ft0, vt0, rt0),
            (f01, v01, e1, r1, ft1, vt1, rt1),
            (f02, v02, e2, r2, ft2, vt2, rt2)):
        v0i = TWO_PI * v0c[...]
        f0i = TWO_PI * (jnp.remainder(f0c[...] + 0.5, 1.0) - 0.5)
        v_t = e * v0i + sigma_v * evc[...]
        mu = _wrap_pi(coeff * (v_t + v0i))
        r_t = _wrap_pi(mu + sig_r * erc[...])
        f_t = _wrap_pi(f0i + r_t)
        ftc[...] = f_t * INV_TWO_PI
        vtc[...] = v_t * INV_TWO_PI
        rtc[...] = r_t * INV_TWO_PI


def _dense(t, planes):
    spec = pl.BlockSpec((_BLK,), lambda i: (i,))
    return pl.pallas_call(
        _dense_body,
        grid=(_GRID,),
        in_specs=[spec] * 13,
        out_specs=[spec] * 9,
        out_shape=[jax.ShapeDtypeStruct((N,), jnp.float32)] * 9,
    )(t, *planes)


def kernel(t, f0, index, v0, epsilon_v, epsilon_r):
    evp = [epsilon_v[:, i] for i in range(3)]
    erp = [epsilon_r[:, i] for i in range(3)]
    f0p = [f0[:, i] for i in range(3)]
    v0p = [v0[:, i] for i in range(3)]
    zeros = jnp.zeros((SP, 8), jnp.float32)
    partials = _accum(*evp, *erp, index, zeros)
    table = _merge(partials)
    cent = _center(table, index, *evp, *erp)
    outs = _dense(t, f0p + v0p + list(cent))
    ft = jnp.stack(outs[0:3], axis=1)
    vt = jnp.stack(outs[3:6], axis=1)
    rt = jnp.stack(outs[6:9], axis=1)
    evc = jnp.stack(cent[0:3], axis=1)
    erc = jnp.stack(cent[3:6], axis=1)
    return (ft, vt, evc, erc, rt)


# speedup vs baseline: 1.0618x; 1.0618x over previous
import functools
import math

import jax
import jax.numpy as jnp
from jax import lax
from jax.experimental import pallas as pl
from jax.experimental.pallas import tpu as pltpu
from jax.experimental.pallas import tpu_sc as plsc

N = 3200000
S = 100000
EPS = 1e-05
PI = math.pi
TWO_PI = 2.0 * math.pi
INV_TWO_PI = 1.0 / TWO_PI

NW = 32
SUB = 128
NSUB = N // SUB
K = 20
CH = K * SUB
NOUT = NSUB // K
PAIRS = NOUT // 2
_P_BASE = PAIRS // NW
_P_REM = PAIRS - _P_BASE * NW
SP = 100096
STRIPE = SP // 16
MSTR = SP // 32

_mesh = plsc.VectorSubcoreMesh(core_axis_name="c", subcore_axis_name="s")
_sc_params = pltpu.CompilerParams(needs_layout_passes=False,
                                  use_tc_tiling_on_sc=False)


def _worker_span(w):
    base = w * _P_BASE + jnp.minimum(w, _P_REM)
    cnt = jnp.where(w < _P_REM, _P_BASE + 1, _P_BASE)
    return base, cnt


def _col(v):
    return jnp.full((16,), v, jnp.int32)


@functools.partial(
    pl.kernel,
    out_type=jax.ShapeDtypeStruct((2, SP, 8), jnp.float32),
    mesh=_mesh,
    scratch_types=[
        pltpu.VMEM((CH,), jnp.int32)] * 2 + [pltpu.VMEM((CH,), jnp.float32)] * 12 + [
        pltpu.VMEM((CH, 8), jnp.float32)] * 2 + [
        pltpu.VMEM_SHARED((SP, 8), jnp.float32),
        pltpu.SemaphoreType.DMA,
        pltpu.SemaphoreType.DMA,
        pltpu.SemaphoreType.DMA,
    ],
    compiler_params=_sc_params,
)
def _accum(ev0, ev1, ev2, er0, er1, er2, idx_hbm, zeros_hbm, pout_hbm,
           idx_a, idx_b, a0, a1, a2, a3, a4, a5, c0, c1, c2, c3, c4, c5,
           aug_a, aug_b, sh, sem_a, sem_b, sem_sc):
    c = lax.axis_index("c")
    s = lax.axis_index("s")
    w = c * 16 + s
    lanes = lax.iota(jnp.int32, 16)
    pltpu.sync_copy(zeros_hbm.at[pl.ds(s * STRIPE, STRIPE)],
                    sh.at[pl.ds(s * STRIPE, STRIPE)])
    for aug in (aug_a, aug_b):
        @pl.loop(0, CH // 16)
        def _(g):
            r = g * 16 + lanes
            plsc.store_scatter(aug, [r, _col(6)], jnp.full((16,), 1.0, jnp.float32))
            plsc.store_scatter(aug, [r, _col(7)], jnp.full((16,), 0.0, jnp.float32))

    plsc.subcore_barrier()
    base, cnt = _worker_span(w)
    bufs_a = (a0, a1, a2, a3, a4, a5)
    bufs_b = (c0, c1, c2, c3, c4, c5)
    srcs = (ev0, ev1, ev2, er0, er1, er2)

    def _assemble(bufs, aug):
        @pl.loop(0, CH // 16)
        def _(g):
            r = g * 16 + lanes
            sl = pl.ds(g * 16, 16)
            for col, b in enumerate(bufs):
                plsc.store_scatter(aug, [r, _col(col)], b[sl])

    @pl.loop(base, base + cnt)
    def _(u):
        row_a = (2 * u) * CH
        row_b = row_a + CH
        sl_a = pl.ds(row_a, CH)
        sl_b = pl.ds(row_b, CH)
        da = [pltpu.async_copy(idx_hbm.at[sl_a], idx_a, sem_a)]
        da += [pltpu.async_copy(src.at[sl_a], b, sem_a)
               for b, src in zip(bufs_a, srcs)]
        db = [pltpu.async_copy(idx_hbm.at[sl_b], idx_b, sem_b)]
        db += [pltpu.async_copy(src.at[sl_b], b, sem_b)
               for b, src in zip(bufs_b, srcs)]
        for d in da:
            d.wait()
        _assemble(bufs_a, aug_a)
        scs = []
        for k in range(K):
            sl_k = pl.ds(k * SUB, SUB)
            scs.append(pltpu.async_copy(aug_a.at[sl_k],
                                        sh.at[idx_a.at[sl_k]],
                                        sem_sc, add=True))
        for d in db:
            d.wait()
        _assemble(bufs_b, aug_b)
        for k in range(K):
            sl_k = pl.ds(k * SUB, SUB)
            scs.append(pltpu.async_copy(aug_b.at[sl_k],
                                        sh.at[idx_b.at[sl_k]],
                                        sem_sc, add=True))
        for d in scs:
            d.wait()

    plsc.subcore_barrier()
    pltpu.sync_copy(sh.at[pl.ds(s * STRIPE, STRIPE)],
                    pout_hbm.at[c].at[pl.ds(s * STRIPE, STRIPE)])


def _merge_body(p_ref, o_ref):
    p = p_ref[...]
    sm = p[0] + p[1]
    cnt = jnp.clip(sm[:, 6:7], 1.0, None)
    o_ref[...] = sm / cnt


def _merge(partials):
    bs = 3128
    return pl.pallas_call(
        _merge_body,
        grid=(SP // bs,),
        in_specs=[pl.BlockSpec((2, bs, 8), lambda i: (0, i, 0))],
        out_specs=pl.BlockSpec((bs, 8), lambda i: (i, 0)),
        out_shape=jax.ShapeDtypeStruct((SP, 8), jnp.float32),
    )(partials)


@functools.partial(
    pl.kernel,
    out_type=[jax.ShapeDtypeStruct((N,), jnp.float32)] * 6,
    mesh=_mesh,
    scratch_types=[
        pltpu.VMEM((CH,), jnp.int32)] * 2 + [pltpu.VMEM((CH,), jnp.float32)] * 12 + [
        pltpu.VMEM((CH, 8), jnp.float32)] * 2 + [
        pltpu.SemaphoreType.DMA,
        pltpu.SemaphoreType.DMA,
        pltpu.SemaphoreType.DMA,
        pltpu.SemaphoreType.DMA,
        pltpu.SemaphoreType.DMA,
    ],
    compiler_params=_sc_params,
)
def _center(table, idx_hbm, ev0, ev1, ev2, er0, er1, er2,
            oev0, oev1, oev2, oer0, oer1, oer2,
            idx_a, idx_b, a0, a1, a2, a3, a4, a5, c0, c1, c2, c3, c4, c5,
            rows_a, rows_b, sem_a, sem_b, sem_ga, sem_gb, sem_out):
    c = lax.axis_index("c")
    s = lax.axis_index("s")
    w = c * 16 + s
    lanes = lax.iota(jnp.int32, 16)
    base, cnt = _worker_span(w)
    bufs_a = (a0, a1, a2, a3, a4, a5)
    bufs_b = (c0, c1, c2, c3, c4, c5)
    srcs = (ev0, ev1, ev2, er0, er1, er2)
    outs = (oev0, oev1, oev2, oer0, oer1, oer2)

    def _subtract(bufs, rows):
        @pl.loop(0, CH // 16)
        def _(g):
            r = g * 16 + lanes
            sl = pl.ds(g * 16, 16)
            for col, b in enumerate(bufs):
                b[sl] = b[sl] - plsc.load_gather(rows, [r, _col(col)])

    @pl.loop(base, base + cnt)
    def _(u):
        row_a = (2 * u) * CH
        row_b = row_a + CH
        sl_a = pl.ds(row_a, CH)
        sl_b = pl.ds(row_b, CH)
        dia = pltpu.async_copy(idx_hbm.at[sl_a], idx_a, sem_a)
        dib = pltpu.async_copy(idx_hbm.at[sl_b], idx_b, sem_b)
        da = [pltpu.async_copy(src.at[sl_a], b, sem_a)
              for b, src in zip(bufs_a, srcs)]
        db = [pltpu.async_copy(src.at[sl_b], b, sem_b)
              for b, src in zip(bufs_b, srcs)]
        dia.wait()
        ga = [pltpu.async_copy(table.at[idx_a.at[pl.ds(k * SUB, SUB)]],
                               rows_a.at[pl.ds(k * SUB, SUB)], sem_ga)
              for k in range(K)]
        dib.wait()
        gb = [pltpu.async_copy(table.at[idx_b.at[pl.ds(k * SUB, SUB)]],
                               rows_b.at[pl.ds(k * SUB, SUB)], sem_gb)
              for k in range(K)]
        for d in da:
            d.wait()
        for d in ga:
            d.wait()
        _subtract(bufs_a, rows_a)
        oda = [pltpu.async_copy(b, dst.at[sl_a], sem_out)
               for b, dst in zip(bufs_a, outs)]
        for d in db:
            d.wait()
        for d in gb:
            d.wait()
        _subtract(bufs_b, rows_b)
        odb = [pltpu.async_copy(b, dst.at[sl_b], sem_out)
               for b, dst in zip(bufs_b, outs)]
        for d in oda:
            d.wait()
        for d in odb:
            d.wait()


_GRID = 25
_BLK = N // _GRID


def _wrap_pi(x):
    return jnp.remainder(x + PI, TWO_PI) - PI


def _dense_body(t_ref, f00, f01, f02, v00, v01, v02, e0, e1, e2, r0, r1, r2,
                ft0, ft1, ft2, vt0, vt1, vt2, rt0, rt1, rt2):
    tt = 2.0 * t_ref[...]
    e = jnp.exp(-tt)
    sigma_v = jnp.sqrt(jnp.clip(1.0 - e * e, EPS, None))
    coeff = (1.0 - e) / (1.0 + e)
    sig_r = jnp.sqrt(jnp.clip(2.0 * tt + 8.0 / (1.0 + jnp.exp(tt)) - 4.0, EPS, None))
    for f0c, v0c, evc, erc, ftc, vtc, rtc in (
            (f00, v00, e0, r0, ft0, vt0, rt0),
            (f01, v01, e1, r1, ft1, vt1, rt1),
            (f02, v02, e2, r2, ft2, vt2, rt2)):
        v0i = TWO_PI * v0c[...]
        f0i = TWO_PI * (jnp.remainder(f0c[...] + 0.5, 1.0) - 0.5)
        v_t = e * v0i + sigma_v * evc[...]
        mu = _wrap_pi(coeff * (v_t + v0i))
        r_t = _wrap_pi(mu + sig_r * erc[...])
        f_t = _wrap_pi(f0i + r_t)
        ftc[...] = f_t * INV_TWO_PI
        vtc[...] = v_t * INV_TWO_PI
        rtc[...] = r_t * INV_TWO_PI


def _dense(t, planes):
    spec = pl.BlockSpec((_BLK,), lambda i: (i,))
    return pl.pallas_call(
        _dense_body,
        grid=(_GRID,),
        in_specs=[spec] * 13,
        out_specs=[spec] * 9,
        out_shape=[jax.ShapeDtypeStruct((N,), jnp.float32)] * 9,
    )(t, *planes)


def kernel(t, f0, index, v0, epsilon_v, epsilon_r):
    evp = [epsilon_v[:, i] for i in range(3)]
    erp = [epsilon_r[:, i] for i in range(3)]
    f0p = [f0[:, i] for i in range(3)]
    v0p = [v0[:, i] for i in range(3)]
    zeros = jnp.zeros((SP, 8), jnp.float32)
    partials = _accum(*evp, *erp, index, zeros)
    table = _merge(partials)
    cent = _center(table, index, *evp, *erp)
    outs = _dense(t, f0p + v0p + list(cent))
    ft = jnp.stack(outs[0:3], axis=1)
    vt = jnp.stack(outs[3:6], axis=1)
    rt = jnp.stack(outs[6:9], axis=1)
    evc = jnp.stack(cent[0:3], axis=1)
    erc = jnp.stack(cent[3:6], axis=1)
    return (ft, vt, evc, erc, rt)
